# baseline (speedup 1.0000x reference)
import functools

import jax
import jax.numpy as jnp
from jax import lax
from jax.experimental import pallas as pl
from jax.experimental.pallas import tpu as pltpu
from jax.experimental.pallas import tpu_sc as plsc

VOCAB = 100000
EMBED = 64
BATCH = 4096

BMS = 64
NSTRIP = BATCH // BMS
BN = 4096
JN = pl.cdiv(VOCAB, BN)
TAIL = VOCAB - (JN - 1) * BN
VPAD = JN * BN


@functools.cache
def _sc_gather():
    info = plsc.get_sparse_core_info()
    nc, ns = info.num_cores, info.num_subcores
    nw = nc * ns
    b_per_w = BATCH // nw
    mesh = plsc.VectorSubcoreMesh(core_axis_name="c", subcore_axis_name="s")

    @functools.partial(
        pl.kernel,
        mesh=mesh,
        out_type=jax.ShapeDtypeStruct((BATCH, EMBED), jnp.float32),
        scratch_types=[
            pltpu.VMEM((b_per_w,), jnp.int32),
            pltpu.VMEM((b_per_w, EMBED), jnp.float32),
            pltpu.SemaphoreType.DMA,
        ],
        compiler_params=pltpu.CompilerParams(use_tc_tiling_on_sc=False),
    )
    def gather(table_hbm, idx_hbm, out_hbm, idx_v, rows_v, sem):
        wid = lax.axis_index("s") * nc + lax.axis_index("c")
        base = wid * b_per_w
        pltpu.sync_copy(idx_hbm.at[pl.ds(base, b_per_w)], idx_v)
        pltpu.async_copy(table_hbm.at[idx_v], rows_v, sem).wait()
        pltpu.sync_copy(rows_v, out_hbm.at[pl.ds(base, b_per_w)])

    return gather


def _strip_copy(acc, out_hbm, sems, s, i):
    return pltpu.make_async_copy(
        acc.at[s], out_hbm.at[pl.ds(i * BMS, BMS)], sems.at[s]
    )


def _mm_body(emb_ref, wt_ref, b_ref, out_hbm, acc, sems):
    i = pl.program_id(0)
    emb = emb_ref[...]

    for s in range(2):
        @pl.when(lax.rem(i, 2) == s)
        def _(s=s):
            @pl.when(i >= 2)
            def _():
                _strip_copy(acc, out_hbm, sems, s, 0).wait()

            for c in range(JN):
                resc = lax.dot_general(
                    emb, wt_ref[:, pl.ds(c * BN, BN)],
                    (((1,), (0,)), ((), ())),
                    preferred_element_type=jnp.float32,
                ) + b_ref[:, pl.ds(c * BN, BN)]
                if c < JN - 1:
                    acc[s, :, pl.ds(c * BN, BN)] = resc
                else:
                    acc[s, :, pl.ds(c * BN, TAIL)] = resc[:, :TAIL]

            _strip_copy(acc, out_hbm, sems, s, i).start(priority=s)

    @pl.when(i == NSTRIP - 1)
    def _():
        for s in range(2):
            _strip_copy(acc, out_hbm, sems, s, 0).wait()


def _tc_matmul(embedded, WT, b):
    return pl.pallas_call(
        _mm_body,
        grid=(NSTRIP,),
        in_specs=[
            pl.BlockSpec((BMS, EMBED), lambda i: (i, 0)),
            pl.BlockSpec((EMBED, VPAD), lambda i: (0, 0)),
            pl.BlockSpec((1, VPAD), lambda i: (0, 0)),
        ],
        out_specs=pl.BlockSpec(memory_space=pl.ANY),
        out_shape=jax.ShapeDtypeStruct((BATCH, VOCAB), jnp.float32),
        scratch_shapes=[
            pltpu.VMEM((2, BMS, VOCAB), jnp.float32),
            pltpu.SemaphoreType.DMA((2,)),
        ],
        compiler_params=pltpu.CompilerParams(
            dimension_semantics=("arbitrary",),
            vmem_limit_bytes=110 * 1024 * 1024,
        ),
    )(embedded, WT, b.reshape(1, VPAD))


def kernel(inputs, emb_table, W, b):
    embedded = _sc_gather()(emb_table, inputs)
    WT = jnp.pad(W.T.astype(jnp.bfloat16), ((0, 0), (0, VPAD - VOCAB)))
    bp = jnp.pad(b, (0, VPAD - VOCAB))
    return _tc_matmul(embedded.astype(jnp.bfloat16), WT, bp)


# speedup vs baseline: 1.0829x; 1.0829x over previous
import functools

import jax
import jax.numpy as jnp
from jax import lax
from jax.experimental import pallas as pl
from jax.experimental.pallas import tpu as pltpu
from jax.experimental.pallas import tpu_sc as plsc

VOCAB = 100000
EMBED = 64
BATCH = 4096

BN = 512


@functools.cache
def _sc_gather():
    info = plsc.get_sparse_core_info()
    nc, ns = info.num_cores, info.num_subcores
    nw = nc * ns
    b_per_w = BATCH // nw
    mesh = plsc.VectorSubcoreMesh(core_axis_name="c", subcore_axis_name="s")

    @functools.partial(
        pl.kernel,
        mesh=mesh,
        out_type=jax.ShapeDtypeStruct((BATCH, EMBED), jnp.float32),
        scratch_types=[
            pltpu.VMEM((b_per_w,), jnp.int32),
            pltpu.VMEM((b_per_w, EMBED), jnp.float32),
            pltpu.SemaphoreType.DMA,
        ],
        compiler_params=pltpu.CompilerParams(use_tc_tiling_on_sc=False),
    )
    def gather(table_hbm, idx_hbm, out_hbm, idx_v, rows_v, sem):
        wid = lax.axis_index("s") * nc + lax.axis_index("c")
        base = wid * b_per_w
        pltpu.sync_copy(idx_hbm.at[pl.ds(base, b_per_w)], idx_v)
        pltpu.async_copy(table_hbm.at[idx_v], rows_v, sem).wait()
        pltpu.sync_copy(rows_v, out_hbm.at[pl.ds(base, b_per_w)])

    return gather


def _mm_body(emb_ref, wt_ref, b_ref, out_ref):
    out_ref[...] = lax.dot_general(
        emb_ref[...], wt_ref[...],
        (((1,), (0,)), ((), ())),
        preferred_element_type=jnp.float32,
    ) + b_ref[...]


def _tc_matmul(embedded, WT, b):
    grid = (pl.cdiv(VOCAB, BN),)
    return pl.pallas_call(
        _mm_body,
        grid=grid,
        in_specs=[
            pl.BlockSpec((BATCH, EMBED), lambda j: (0, 0)),
            pl.BlockSpec((EMBED, BN), lambda j: (0, j)),
            pl.BlockSpec((1, BN), lambda j: (0, j)),
        ],
        out_specs=pl.BlockSpec((BATCH, BN), lambda j: (0, j)),
        out_shape=jax.ShapeDtypeStruct((BATCH, VOCAB), jnp.float32),
        compiler_params=pltpu.CompilerParams(
            dimension_semantics=("arbitrary",),
        ),
    )(embedded, WT, b.reshape(1, VOCAB))


def kernel(inputs, emb_table, W, b):
    embedded = _sc_gather()(emb_table, inputs)
    WT = W.T.astype(jnp.bfloat16)
    return _tc_matmul(embedded.astype(jnp.bfloat16), WT, b)
